# baseline (speedup 1.0000x reference)
import jax
import jax.numpy as jnp
from jax.experimental import pallas as pl
from jax.experimental.pallas import tpu as pltpu

B, P, C, G = 32, 8732, 21, 12
IMG = 8
THRESH = 0.5
NEG_POS = 3
VAR0, VAR1 = 0.1, 0.2


def _match_kernel(priors_ref, targets_ref,
                  ct_ref, k_ref, g4_ref, np_ref):
    t = targets_ref[...]
    tx1 = t[:, :, 0:1]
    ty1 = t[:, :, 1:2]
    tx2 = t[:, :, 2:3]
    ty2 = t[:, :, 3:4]

    pr = priors_ref[...]
    pcx = pr[0:1, :]
    pcy = pr[1:2, :]
    pw = pr[2:3, :]
    ph = pr[3:4, :]
    px1 = (pcx - pw * 0.5)[None]
    py1 = (pcy - ph * 0.5)[None]
    px2 = (pcx + pw * 0.5)[None]
    py2 = (pcy + ph * 0.5)[None]

    iw = jnp.maximum(jnp.minimum(tx2, px2) - jnp.maximum(tx1, px1), 0.0)
    ih = jnp.maximum(jnp.minimum(ty2, py2) - jnp.maximum(ty1, py1), 0.0)
    inter = iw * ih
    area_t = (tx2 - tx1) * (ty2 - ty1)
    area_p = (px2 - px1) * (py2 - py1)
    ov = inter / (area_t + area_p - inter)

    g_iota = jax.lax.broadcasted_iota(jnp.int32, (1, G, 1), 1)
    col_iota3 = jax.lax.broadcasted_iota(jnp.int32, (1, 1, P), 2)

    bto = jnp.max(ov, axis=1, keepdims=True)
    bti = jnp.min(jnp.where(ov == bto, g_iota, G), axis=1)

    rowmax = jnp.max(ov, axis=2, keepdims=True)
    bpi = jnp.min(jnp.where(ov == rowmax, col_iota3, P), axis=2,
                  keepdims=True)

    hit = col_iota3 == bpi
    last_g = jnp.max(jnp.where(hit, g_iota, -1), axis=1)
    forced = last_g >= 0
    bti = jnp.where(forced, last_g, bti)
    bto2 = jnp.where(forced, 2.0, bto[:, 0, :])

    sel = (g_iota == bti[:, None, :]).astype(jnp.float32)
    matched = jax.lax.dot_general(
        t, sel, dimension_numbers=(((1,), (1,)), ((0,), (0,))),
        preferred_element_type=jnp.float32,
        precision=jax.lax.Precision.HIGHEST)
    mx1 = matched[:, 0, :]
    my1 = matched[:, 1, :]
    mx2 = matched[:, 2, :]
    my2 = matched[:, 3, :]
    mlab = matched[:, 4, :]

    conf_t = jnp.where(bto2 < THRESH, 0,
                       jnp.round(mlab).astype(jnp.int32) + 1)
    posf = (conf_t > 0).astype(jnp.float32)
    npos = jnp.sum(posf, axis=1, keepdims=True)

    pcx2, pcy2, pw2, ph2 = pr[0:1], pr[1:2], pr[2:3], pr[3:4]
    g_cx = ((mx1 + mx2) * 0.5 - pcx2) / (VAR0 * pw2)
    g_cy = ((my1 + my2) * 0.5 - pcy2) / (VAR0 * ph2)
    g_w = jnp.log((mx2 - mx1) / pw2) / VAR1
    g_h = jnp.log((my2 - my1) / ph2) / VAR1

    g4_ref[...] = jnp.stack([g_cx, g_cy, g_w, g_h], axis=1)
    ct_ref[...] = conf_t
    k_ref[...] = jnp.minimum(
        jnp.float32(NEG_POS) * npos, jnp.float32(P - 1)).astype(jnp.int32)
    np_ref[...] = npos


def _ce_kernel(conf_ref, loc_ref, ct_ref, g4_ref, s_ref, lcp_ref, ll_ref):
    conf_t = ct_ref[...]
    pos = conf_t > 0
    posf = pos.astype(jnp.float32)

    d = loc_ref[...] - g4_ref[...]
    a = jnp.abs(d)
    sl1 = jnp.where(a < 1.0, 0.5 * a * a, a - 0.5)
    ll_ref[...] = jnp.sum(jnp.sum(sl1, axis=1) * posf,
                          axis=1, keepdims=True)

    x = conf_ref[...]
    m = jnp.max(x, axis=1, keepdims=True)
    lse = jnp.log(jnp.sum(jnp.exp(x - m), axis=1)) + m[:, 0, :]
    cls_iota = jax.lax.broadcasted_iota(jnp.int32, (1, C, 1), 1)
    onehot = (cls_iota == conf_t[:, None, :]).astype(jnp.float32)
    gathered = jnp.sum(onehot * x, axis=1)
    ce = lse - gathered
    r = jnp.where(pos, 0.0, ce)

    s_ref[...] = jax.lax.bitcast_convert_type(r, jnp.int32)
    lcp_ref[...] = jnp.sum(ce * posf, axis=1, keepdims=True)


def _select_kernel(s_ref, k_ref, ll_ref, lcp_ref, np_ref,
                   out_l_ref, out_c_ref):
    s = s_ref[...]
    kv = k_ref[...]
    col_iota = jax.lax.broadcasted_iota(jnp.int32, (1, P), 1)

    def find_t(b, acc):
        cand = acc | (jnp.int32(1) << (30 - b))
        cnt = jnp.sum((s >= cand).astype(jnp.int32), axis=1, keepdims=True)
        return jnp.where(cnt >= kv, cand, acc)

    tv = jax.lax.fori_loop(0, 31, find_t, jnp.zeros((B, 1), jnp.int32))

    cnt_gt = jnp.sum((s > tv).astype(jnp.int32), axis=1, keepdims=True)
    need = kv - cnt_gt
    tie = s == tv

    def find_m(b, acc):
        cand = acc | (jnp.int32(1) << (13 - b))
        cnt = jnp.sum((tie & (col_iota < cand)).astype(jnp.int32),
                      axis=1, keepdims=True)
        return jnp.where(cnt <= need, cand, acc)

    mv = jax.lax.fori_loop(0, 14, find_m, jnp.zeros((B, 1), jnp.int32))

    neg = (s > tv) | (tie & (col_iota < mv))
    cez = jax.lax.bitcast_convert_type(jnp.maximum(s, 0), jnp.float32)
    lc_neg = jnp.sum(cez * neg.astype(jnp.float32))

    nf = jnp.sum(np_ref[...])
    out_l_ref[0, 0] = jnp.sum(ll_ref[...]) / nf
    out_c_ref[0, 0] = (jnp.sum(lcp_ref[...]) + lc_neg) / nf


def kernel(loc_data, conf_data, priors, targets):
    loc_t = jnp.transpose(loc_data, (0, 2, 1))
    conf_tr = jnp.transpose(conf_data, (0, 2, 1))
    priors_t = jnp.transpose(priors, (1, 0))

    nblk = B // IMG
    ct, k, g4, npv = pl.pallas_call(
        _match_kernel,
        grid=(nblk,),
        in_specs=[
            pl.BlockSpec((4, P), lambda b: (0, 0)),
            pl.BlockSpec((IMG, G, 5), lambda b: (b, 0, 0)),
        ],
        out_specs=[
            pl.BlockSpec((IMG, P), lambda b: (b, 0)),
            pl.BlockSpec((IMG, 1), lambda b: (b, 0)),
            pl.BlockSpec((IMG, 4, P), lambda b: (b, 0, 0)),
            pl.BlockSpec((IMG, 1), lambda b: (b, 0)),
        ],
        out_shape=[
            jax.ShapeDtypeStruct((B, P), jnp.int32),
            jax.ShapeDtypeStruct((B, 1), jnp.int32),
            jax.ShapeDtypeStruct((B, 4, P), jnp.float32),
            jax.ShapeDtypeStruct((B, 1), jnp.float32),
        ],
        compiler_params=pltpu.CompilerParams(
            dimension_semantics=("parallel",)),
    )(priors_t, targets)

    s, lcp, ll = pl.pallas_call(
        _ce_kernel,
        grid=(nblk,),
        in_specs=[
            pl.BlockSpec((IMG, C, P), lambda b: (b, 0, 0)),
            pl.BlockSpec((IMG, 4, P), lambda b: (b, 0, 0)),
            pl.BlockSpec((IMG, P), lambda b: (b, 0)),
            pl.BlockSpec((IMG, 4, P), lambda b: (b, 0, 0)),
        ],
        out_specs=[
            pl.BlockSpec((IMG, P), lambda b: (b, 0)),
            pl.BlockSpec((IMG, 1), lambda b: (b, 0)),
            pl.BlockSpec((IMG, 1), lambda b: (b, 0)),
        ],
        out_shape=[
            jax.ShapeDtypeStruct((B, P), jnp.int32),
            jax.ShapeDtypeStruct((B, 1), jnp.float32),
            jax.ShapeDtypeStruct((B, 1), jnp.float32),
        ],
        compiler_params=pltpu.CompilerParams(
            dimension_semantics=("parallel",)),
    )(conf_tr, loc_t, ct, g4)

    scalar_spec = pl.BlockSpec((1, 1), lambda: (0, 0),
                               memory_space=pltpu.SMEM)
    out_l, out_c = pl.pallas_call(
        _select_kernel,
        out_specs=[scalar_spec, scalar_spec],
        out_shape=[jax.ShapeDtypeStruct((1, 1), jnp.float32)] * 2,
    )(s, k, ll, lcp, npv)

    return (out_l[0, 0], out_c[0, 0])


# speedup vs baseline: 1.0210x; 1.0210x over previous
import jax
import jax.numpy as jnp
from jax.experimental import pallas as pl
from jax.experimental.pallas import tpu as pltpu

B, P, C, G = 32, 8732, 21, 12
IMG = 8
THRESH = 0.5
NEG_POS = 3
VAR0, VAR1 = 0.1, 0.2


def _match_kernel(loc_ref, priors_ref, targets_ref,
                  ct_ref, k_ref, ll_ref, np_ref):
    t = targets_ref[...]
    tx1 = t[:, :, 0:1]
    ty1 = t[:, :, 1:2]
    tx2 = t[:, :, 2:3]
    ty2 = t[:, :, 3:4]

    pr = priors_ref[...]
    pcx = pr[0:1, :]
    pcy = pr[1:2, :]
    pw = pr[2:3, :]
    ph = pr[3:4, :]
    px1 = (pcx - pw * 0.5)[None]
    py1 = (pcy - ph * 0.5)[None]
    px2 = (pcx + pw * 0.5)[None]
    py2 = (pcy + ph * 0.5)[None]

    iw = jnp.maximum(jnp.minimum(tx2, px2) - jnp.maximum(tx1, px1), 0.0)
    ih = jnp.maximum(jnp.minimum(ty2, py2) - jnp.maximum(ty1, py1), 0.0)
    inter = iw * ih
    area_t = (tx2 - tx1) * (ty2 - ty1)
    area_p = (px2 - px1) * (py2 - py1)
    ov = inter / (area_t + area_p - inter)

    g_iota = jax.lax.broadcasted_iota(jnp.int32, (1, G, 1), 1)
    col_iota3 = jax.lax.broadcasted_iota(jnp.int32, (1, 1, P), 2)

    bto = jnp.max(ov, axis=1, keepdims=True)
    bti = jnp.min(jnp.where(ov == bto, g_iota, G), axis=1)

    rowmax = jnp.max(ov, axis=2, keepdims=True)
    bpi = jnp.min(jnp.where(ov == rowmax, col_iota3, P), axis=2,
                  keepdims=True)

    hit = col_iota3 == bpi
    last_g = jnp.max(jnp.where(hit, g_iota, -1), axis=1)
    forced = last_g >= 0
    bti = jnp.where(forced, last_g, bti)
    bto2 = jnp.where(forced, 2.0, bto[:, 0, :])

    sel = (g_iota == bti[:, None, :]).astype(jnp.float32)
    matched = jax.lax.dot_general(
        t, sel, dimension_numbers=(((1,), (1,)), ((0,), (0,))),
        preferred_element_type=jnp.float32,
        precision=jax.lax.Precision.HIGHEST)
    mx1 = matched[:, 0, :]
    my1 = matched[:, 1, :]
    mx2 = matched[:, 2, :]
    my2 = matched[:, 3, :]
    mlab = matched[:, 4, :]

    conf_t = jnp.where(bto2 < THRESH, 0,
                       jnp.round(mlab).astype(jnp.int32) + 1)
    posf = (conf_t > 0).astype(jnp.float32)
    npos = jnp.sum(posf, axis=1, keepdims=True)

    pcx2, pcy2, pw2, ph2 = pr[0:1], pr[1:2], pr[2:3], pr[3:4]
    g_cx = ((mx1 + mx2) * 0.5 - pcx2) / (VAR0 * pw2)
    g_cy = ((my1 + my2) * 0.5 - pcy2) / (VAR0 * ph2)
    g_w = jnp.log((mx2 - mx1) / pw2) / VAR1
    g_h = jnp.log((my2 - my1) / ph2) / VAR1

    ld = loc_ref[...]
    d0 = ld[:, 0, :] - g_cx
    d1 = ld[:, 1, :] - g_cy
    d2 = ld[:, 2, :] - g_w
    d3 = ld[:, 3, :] - g_h

    def sl1(d):
        a = jnp.abs(d)
        return jnp.where(a < 1.0, 0.5 * a * a, a - 0.5)

    ll_ref[...] = jnp.sum((sl1(d0) + sl1(d1) + sl1(d2) + sl1(d3)) * posf,
                          axis=1, keepdims=True)

    ct_ref[...] = conf_t
    k_ref[...] = jnp.minimum(
        jnp.float32(NEG_POS) * npos, jnp.float32(P - 1)).astype(jnp.int32)
    np_ref[...] = npos


def _ce_kernel(conf_ref, ct_ref, s_ref, lcp_ref):
    conf_t = ct_ref[...]
    pos = conf_t > 0
    posf = pos.astype(jnp.float32)

    x = conf_ref[...]
    m = jnp.max(x, axis=1, keepdims=True)
    lse = jnp.log(jnp.sum(jnp.exp(x - m), axis=1)) + m[:, 0, :]
    cls_iota = jax.lax.broadcasted_iota(jnp.int32, (1, C, 1), 1)
    onehot = (cls_iota == conf_t[:, None, :]).astype(jnp.float32)
    gathered = jnp.sum(onehot * x, axis=1)
    ce = lse - gathered
    r = jnp.where(pos, 0.0, ce)

    s_ref[...] = jax.lax.bitcast_convert_type(r, jnp.int32)
    lcp_ref[...] = jnp.sum(ce * posf, axis=1, keepdims=True)


def _select_kernel(s_ref, k_ref, ll_ref, lcp_ref, np_ref,
                   out_l_ref, out_c_ref):
    s = s_ref[...]
    kv = k_ref[...]
    col_iota = jax.lax.broadcasted_iota(jnp.int32, (1, P), 1)

    def find_t(b, acc):
        cand = acc | (jnp.int32(1) << (30 - b))
        cnt = jnp.sum((s >= cand).astype(jnp.int32), axis=1, keepdims=True)
        return jnp.where(cnt >= kv, cand, acc)

    tv = jax.lax.fori_loop(0, 31, find_t, jnp.zeros((B, 1), jnp.int32))

    cnt_gt = jnp.sum((s > tv).astype(jnp.int32), axis=1, keepdims=True)
    need = kv - cnt_gt
    tie = s == tv

    def find_m(b, acc):
        cand = acc | (jnp.int32(1) << (13 - b))
        cnt = jnp.sum((tie & (col_iota < cand)).astype(jnp.int32),
                      axis=1, keepdims=True)
        return jnp.where(cnt <= need, cand, acc)

    mv = jax.lax.fori_loop(0, 14, find_m, jnp.zeros((B, 1), jnp.int32))

    neg = (s > tv) | (tie & (col_iota < mv))
    cez = jax.lax.bitcast_convert_type(jnp.maximum(s, 0), jnp.float32)
    lc_neg = jnp.sum(cez * neg.astype(jnp.float32))

    nf = jnp.sum(np_ref[...])
    out_l_ref[0, 0] = jnp.sum(ll_ref[...]) / nf
    out_c_ref[0, 0] = (jnp.sum(lcp_ref[...]) + lc_neg) / nf


def kernel(loc_data, conf_data, priors, targets):
    loc_t = jnp.transpose(loc_data, (0, 2, 1))
    conf_tr = jnp.transpose(conf_data, (0, 2, 1))
    priors_t = jnp.transpose(priors, (1, 0))

    nblk = B // IMG
    ct, k, ll, npv = pl.pallas_call(
        _match_kernel,
        grid=(nblk,),
        in_specs=[
            pl.BlockSpec((IMG, 4, P), lambda b: (b, 0, 0)),
            pl.BlockSpec((4, P), lambda b: (0, 0)),
            pl.BlockSpec((IMG, G, 5), lambda b: (b, 0, 0)),
        ],
        out_specs=[
            pl.BlockSpec((IMG, P), lambda b: (b, 0)),
            pl.BlockSpec((IMG, 1), lambda b: (b, 0)),
            pl.BlockSpec((IMG, 1), lambda b: (b, 0)),
            pl.BlockSpec((IMG, 1), lambda b: (b, 0)),
        ],
        out_shape=[
            jax.ShapeDtypeStruct((B, P), jnp.int32),
            jax.ShapeDtypeStruct((B, 1), jnp.int32),
            jax.ShapeDtypeStruct((B, 1), jnp.float32),
            jax.ShapeDtypeStruct((B, 1), jnp.float32),
        ],
        compiler_params=pltpu.CompilerParams(
            dimension_semantics=("parallel",)),
    )(loc_t, priors_t, targets)

    s, lcp = pl.pallas_call(
        _ce_kernel,
        grid=(nblk,),
        in_specs=[
            pl.BlockSpec((IMG, C, P), lambda b: (b, 0, 0)),
            pl.BlockSpec((IMG, P), lambda b: (b, 0)),
        ],
        out_specs=[
            pl.BlockSpec((IMG, P), lambda b: (b, 0)),
            pl.BlockSpec((IMG, 1), lambda b: (b, 0)),
        ],
        out_shape=[
            jax.ShapeDtypeStruct((B, P), jnp.int32),
            jax.ShapeDtypeStruct((B, 1), jnp.float32),
        ],
        compiler_params=pltpu.CompilerParams(
            dimension_semantics=("parallel",)),
    )(conf_tr, ct)

    scalar_spec = pl.BlockSpec((1, 1), lambda: (0, 0),
                               memory_space=pltpu.SMEM)
    out_l, out_c = pl.pallas_call(
        _select_kernel,
        out_specs=[scalar_spec, scalar_spec],
        out_shape=[jax.ShapeDtypeStruct((1, 1), jnp.float32)] * 2,
    )(s, k, ll, lcp, npv)

    return (out_l[0, 0], out_c[0, 0])
